# baseline (speedup 1.0000x reference)
import functools

import jax
import jax.numpy as jnp
from jax import lax
from jax.experimental import pallas as pl
from jax.experimental.pallas import tpu as pltpu
from jax.experimental.pallas import tpu_sc as plsc

BATCH = 16384
FEAT_DIM = 64
OUT_DIM = 128

_GATHER_CHUNK = 128
_LANES = 16


@functools.cache
def _make_sc_gather(batch, width):
    info = plsc.get_sparse_core_info()
    nw = info.num_cores * info.num_subcores
    bpw = batch // nw
    nchunks = bpw // _GATHER_CHUNK
    mesh = plsc.VectorSubcoreMesh(core_axis_name="c", subcore_axis_name="s")

    @functools.partial(
        pl.kernel,
        mesh=mesh,
        out_type=jax.ShapeDtypeStruct((batch, width), jnp.float32),
        scratch_types=[
            pltpu.VMEM((bpw,), jnp.int32),
            pltpu.VMEM((bpw,), jnp.int32),
            pltpu.VMEM((bpw, width), jnp.float32),
            pltpu.SemaphoreType.DMA,
        ],
    )
    def gather(idx_hbm, table_hbm, out_hbm, idx_v, idx2_v, rows_v, sem):
        wid = lax.axis_index("s") * info.num_cores + lax.axis_index("c")
        base = wid * bpw
        pltpu.sync_copy(idx_hbm.at[pl.ds(base, bpw)], idx_v)
        for j in range(bpw // _LANES):
            sl = pl.ds(j * _LANES, _LANES)
            idx2_v[sl] = lax.shift_right_logical(idx_v[sl], 1)
        copies = []
        for j in range(nchunks):
            sl = pl.ds(j * _GATHER_CHUNK, _GATHER_CHUNK)
            copies.append(
                pltpu.async_copy(table_hbm.at[idx2_v.at[sl]], rows_v.at[sl], sem)
            )
        for c in copies:
            c.wait()
        pltpu.sync_copy(rows_v, out_hbm.at[pl.ds(base, bpw)])

    return gather


def _layernorm(x, eps=1e-5):
    mu = jnp.mean(x, axis=-1, keepdims=True)
    var = jnp.mean((x - mu) ** 2, axis=-1, keepdims=True)
    return (x - mu) / jnp.sqrt(var + eps)


def _gelu_exact(x):
    return 0.5 * x * (1.0 + lax.erf(x * 0.7071067811865476))


def _mlp_body(pairs_ref, ids_ref, w1_ref, b1_ref, w2_ref, b2_ref, w3_ref, b3_ref, out_ref):
    pairs = pairs_ref[...]
    odd = (ids_ref[...] & 1) == 1
    x = jnp.where(odd, pairs[:, FEAT_DIM:], pairs[:, :FEAT_DIM])
    h = jnp.dot(x, w1_ref[...], preferred_element_type=jnp.float32) + b1_ref[...]
    h = _gelu_exact(_layernorm(h))
    h = jnp.dot(h, w2_ref[...], preferred_element_type=jnp.float32) + b2_ref[...]
    h = _gelu_exact(_layernorm(h))
    h = jnp.dot(h, w3_ref[...], preferred_element_type=jnp.float32) + b3_ref[...]
    out_ref[...] = _gelu_exact(h)


def _tc_mlp(pairs, ids, w1, b1, w2, b2, w3, b3, blk=2048, interpret=False):
    batch = pairs.shape[0]
    grid = (batch // blk,)
    rep2 = lambda i: (0, 0)
    return pl.pallas_call(
        _mlp_body,
        grid=grid,
        in_specs=[
            pl.BlockSpec((blk, pairs.shape[1]), lambda i: (i, 0)),
            pl.BlockSpec((blk, 1), lambda i: (i, 0)),
            pl.BlockSpec(w1.shape, rep2),
            pl.BlockSpec(b1.shape, rep2),
            pl.BlockSpec(w2.shape, rep2),
            pl.BlockSpec(b2.shape, rep2),
            pl.BlockSpec(w3.shape, rep2),
            pl.BlockSpec(b3.shape, rep2),
        ],
        out_specs=pl.BlockSpec((blk, w3.shape[1]), lambda i: (i, 0)),
        out_shape=jax.ShapeDtypeStruct((batch, w3.shape[1]), jnp.float32),
        interpret=interpret,
    )(pairs, ids, w1, b1, w2, b2, w3, b3)


def kernel(user_ids, table, W1, b1, W2, b2, W3, b3):
    idx = user_ids.astype(jnp.int32)
    table_pairs = table.reshape(table.shape[0] // 2, 2 * table.shape[1])
    pairs = _make_sc_gather(BATCH, 2 * FEAT_DIM)(idx, table_pairs)
    return _tc_mlp(
        pairs,
        idx.reshape(-1, 1),
        W1,
        b1.reshape(1, -1),
        W2,
        b2.reshape(1, -1),
        W3,
        b3.reshape(1, -1),
    )


# speedup vs baseline: 2.8884x; 2.8884x over previous
import functools

import jax
import jax.numpy as jnp
from jax import lax
from jax.experimental import pallas as pl
from jax.experimental.pallas import tpu as pltpu
from jax.experimental.pallas import tpu_sc as plsc

BATCH = 16384
FEAT_DIM = 64
NUM_USERS = 1000000

_NW = 32
_K = 16
_CHW = _K * 128
_TPW = 245
_UPW = _TPW * 128
_NCH = 16
_FULL_U = 999936
_LAST_U0 = _FULL_U - _CHW
_BCAP = 256
_I16 = lambda: lax.iota(jnp.int32, 16)


@functools.cache
def _make_sc_scan():
    mesh = plsc.VectorSubcoreMesh(core_axis_name="c", subcore_axis_name="s")
    nc = 2

    @functools.partial(
        pl.kernel,
        mesh=mesh,
        out_type=jax.ShapeDtypeStruct((BATCH, 128), jnp.float32),
        scratch_types=[
            pltpu.VMEM((BATCH,), jnp.int32),
            pltpu.VMEM((BATCH,), jnp.int32),
            pltpu.VMEM((2048,), jnp.int32),
            pltpu.VMEM((8, _CHW), jnp.float32),
            pltpu.VMEM((8, _CHW), jnp.float32),
            pltpu.VMEM((_BCAP, 128), jnp.float32),
            pltpu.VMEM((_BCAP,), jnp.int32),
            pltpu.VMEM((_BCAP,), jnp.int32),
            pltpu.VMEM((1, _BCAP), jnp.int32),
            pltpu.VMEM((64, 64), jnp.float32),
            pltpu.SemaphoreType.DMA,
            pltpu.SemaphoreType.DMA,
            pltpu.SemaphoreType.DMA,
        ],
        compiler_params=pltpu.CompilerParams(needs_layout_passes=False),
    )
    def scan(idx_hbm, t3_hbm, tail_hbm, out_hbm, wval, wpos, idxbuf, slab_a,
             slab_b, emb_v, bktv, bktp, bktp2, tail_v, sem_a, sem_b, sem_s):
        wid = lax.axis_index("s") * nc + lax.axis_index("c")
        lo = wid * _UPW
        hi = jnp.where(wid == _NW - 1, NUM_USERS, lo + _UPW)
        lanes = _I16()

        def _piece(p, cnt):
            pltpu.sync_copy(idx_hbm.at[pl.ds(p * 2048, 2048)], idxbuf)

            def _vstep(g, cnt):
                off = pl.multiple_of(g * 16, 16)
                v = idxbuf[pl.ds(off, 16)]
                m = (v >= lo) & (v < hi)
                cs = plsc.cumsum(m.astype(jnp.int32))
                tgt = cnt + cs - 1
                plsc.store_scatter(wval, [tgt], v, mask=m)
                plsc.store_scatter(
                    wpos, [tgt], p * 2048 + off + lanes, mask=m
                )
                return cnt + jnp.sum(m.astype(jnp.int32))

            return lax.fori_loop(0, 128, _vstep, cnt)

        n_w = jnp.int32(0)
        for p in range(8):
            n_w = _piece(p, n_w)
        nv = (n_w + 15) // 16

        def _count_range(rlo, rhi):
            def _vstep(g, r):
                off = pl.multiple_of(g * 16, 16)
                v = wval[pl.ds(off, 16)]
                m = (v >= rlo) & (v < rhi)
                return r + jnp.sum(m.astype(jnp.int32))

            return lax.fori_loop(0, nv, _vstep, jnp.int32(0))

        def _fill_window(rlo, rhi, blk):
            for j in range(_BCAP // 16):
                bktv[pl.ds(j * 16, 16)] = jnp.full((16,), 0, jnp.int32) + rlo
                bktp[pl.ds(j * 16, 16)] = jnp.full((16,), -1, jnp.int32)
            wstart = blk * _BCAP

            def _vstep(g, carry):
                r, filled = carry
                off = pl.multiple_of(g * 16, 16)
                v = wval[pl.ds(off, 16)]
                pp = wpos[pl.ds(off, 16)]
                m = (v >= rlo) & (v < rhi)
                cs = plsc.cumsum(m.astype(jnp.int32))
                order = r + cs - 1
                keep = m & (order >= wstart) & (order < wstart + _BCAP)
                tgt = order - wstart
                plsc.store_scatter(bktv, [tgt], v, mask=keep)
                plsc.store_scatter(bktp, [tgt], pp, mask=keep)
                return (
                    r + jnp.sum(m.astype(jnp.int32)),
                    filled + jnp.sum(keep.astype(jnp.int32)),
                )

            _, filled = lax.fori_loop(0, nv, _vstep, (jnp.int32(0), jnp.int32(0)))
            for j in range(_BCAP // 16):
                bktp2[0, pl.ds(j * 16, 16)] = bktp[pl.ds(j * 16, 16)]
            return filled

        def _scatter_out():
            pltpu.async_copy(
                emb_v,
                out_hbm.at[plsc.Indices(bktp2.at[0], ignored_value=-1)],
                sem_s,
            ).wait()

        def _process_slabs(u0, filled):
            ng = (filled + 15) // 16
            cp = pltpu.async_copy(
                t3_hbm.at[0, :, pl.ds(u0, _CHW)], slab_a, sem_a
            )
            for a in range(8):
                buf = slab_a if a % 2 == 0 else slab_b
                if a < 7:
                    nxt_buf = slab_b if a % 2 == 0 else slab_a
                    nxt = pltpu.async_copy(
                        t3_hbm.at[a + 1, :, pl.ds(u0, _CHW)],
                        nxt_buf,
                        sem_b if a % 2 == 0 else sem_a,
                    )
                cp.wait()

                def _estep(g, _, buf=buf, a=a):
                    off = pl.multiple_of(g * 16, 16)
                    uv = bktv[pl.ds(off, 16)] - u0
                    sl = off + lanes
                    for b in range(8):
                        vals = plsc.load_gather(
                            buf, [jnp.full((16,), b, jnp.int32), uv]
                        )
                        plsc.store_scatter(
                            emb_v,
                            [sl, jnp.full((16,), a * 8 + b, jnp.int32)],
                            vals,
                        )
                    return 0

                lax.fori_loop(0, ng, _estep, 0)
                if a < 7:
                    cp = nxt

        def _chunk(c, _):
            u0 = jnp.minimum(lo + c * _CHW, _LAST_U0)
            m_c = _count_range(u0, u0 + _CHW)
            nblk = (m_c + _BCAP - 1) // _BCAP

            def _blk(blk, _):
                filled = _fill_window(u0, u0 + _CHW, blk)
                _process_slabs(u0, filled)
                _scatter_out()
                return 0

            return lax.fori_loop(0, jnp.maximum(nblk, 1), _blk, 0)

        lax.fori_loop(0, _NCH, _chunk, jnp.int32(0))

        @pl.when(wid == _NW - 1)
        def _tail():
            pltpu.sync_copy(tail_hbm, tail_v)
            m_t = _count_range(_FULL_U, NUM_USERS)
            nblk = (m_t + _BCAP - 1) // _BCAP

            def _blk(blk, _):
                filled = _fill_window(_FULL_U, NUM_USERS, blk)
                ng = (filled + 15) // 16

                def _estep(g, _):
                    off = pl.multiple_of(g * 16, 16)
                    uv = bktv[pl.ds(off, 16)] - _FULL_U
                    sl = off + lanes
                    for f in range(FEAT_DIM):
                        vals = plsc.load_gather(
                            tail_v, [uv, jnp.full((16,), f, jnp.int32)]
                        )
                        plsc.store_scatter(
                            emb_v, [sl, jnp.full((16,), f, jnp.int32)], vals
                        )
                    return 0

                lax.fori_loop(0, ng, _estep, 0)
                _scatter_out()
                return 0

            lax.fori_loop(0, nblk, _blk, 0)

    return scan


def _layernorm(x, eps=1e-5):
    mu = jnp.mean(x, axis=-1, keepdims=True)
    var = jnp.mean((x - mu) ** 2, axis=-1, keepdims=True)
    return (x - mu) / jnp.sqrt(var + eps)


def _gelu_exact(x):
    return 0.5 * x * (1.0 + lax.erf(x * 0.7071067811865476))


def _mlp_body(emb_ref, w1_ref, b1_ref, w2_ref, b2_ref, w3_ref, b3_ref, out_ref):
    x = emb_ref[...][:, :FEAT_DIM]
    h = jnp.dot(x, w1_ref[...], preferred_element_type=jnp.float32) + b1_ref[...]
    h = _gelu_exact(_layernorm(h))
    h = jnp.dot(h, w2_ref[...], preferred_element_type=jnp.float32) + b2_ref[...]
    h = _gelu_exact(_layernorm(h))
    h = jnp.dot(h, w3_ref[...], preferred_element_type=jnp.float32) + b3_ref[...]
    out_ref[...] = _gelu_exact(h)


def _tc_mlp(emb, w1, b1, w2, b2, w3, b3, blk=2048, interpret=False):
    batch = emb.shape[0]
    grid = (batch // blk,)
    rep2 = lambda i: (0, 0)
    return pl.pallas_call(
        _mlp_body,
        grid=grid,
        in_specs=[
            pl.BlockSpec((blk, emb.shape[1]), lambda i: (i, 0)),
            pl.BlockSpec(w1.shape, rep2),
            pl.BlockSpec(b1.shape, rep2),
            pl.BlockSpec(w2.shape, rep2),
            pl.BlockSpec(b2.shape, rep2),
            pl.BlockSpec(w3.shape, rep2),
            pl.BlockSpec(b3.shape, rep2),
        ],
        out_specs=pl.BlockSpec((blk, w3.shape[1]), lambda i: (i, 0)),
        out_shape=jax.ShapeDtypeStruct((batch, w3.shape[1]), jnp.float32),
        interpret=interpret,
    )(emb, w1, b1, w2, b2, w3, b3)


def kernel(user_ids, table, W1, b1, W2, b2, W3, b3):
    idx = user_ids.astype(jnp.int32)
    t3 = table.T.reshape(8, 8, NUM_USERS)
    tail = lax.slice(table, (_FULL_U, 0), (NUM_USERS, FEAT_DIM))
    emb = _make_sc_scan()(idx, t3, tail)
    return _tc_mlp(
        emb,
        W1,
        b1.reshape(1, -1),
        W2,
        b2.reshape(1, -1),
        W3,
        b3.reshape(1, -1),
    )


# speedup vs baseline: 3.4022x; 1.1779x over previous
import functools

import jax
import jax.numpy as jnp
from jax import lax
from jax.experimental import pallas as pl
from jax.experimental.pallas import tpu as pltpu
from jax.experimental.pallas import tpu_sc as plsc

BATCH = 16384
FEAT_DIM = 64
NUM_USERS = 1000000

_NW = 32
_K = 32
_CHW = _K * 128
_TPW = 245
_UPW = _TPW * 128
_NCH = 8
_FULL_U = 999936
_LAST_U0 = _FULL_U - _CHW
_BCAP = 128
_POSB = 14
_I16 = lambda: lax.iota(jnp.int32, 16)


@functools.cache
def _make_sc_scan():
    mesh = plsc.VectorSubcoreMesh(core_axis_name="c", subcore_axis_name="s")
    nc = 2

    @functools.partial(
        pl.kernel,
        mesh=mesh,
        out_type=jax.ShapeDtypeStruct((BATCH, 128), jnp.float32),
        scratch_types=[
            pltpu.VMEM((BATCH,), jnp.int32),
            pltpu.VMEM((2048,), jnp.int32),
            pltpu.VMEM((8, _CHW), jnp.float32),
            pltpu.VMEM((8, _CHW), jnp.float32),
            pltpu.VMEM((_BCAP, 128), jnp.float32),
            pltpu.VMEM((_BCAP,), jnp.int32),
            pltpu.VMEM((_BCAP,), jnp.int32),
            pltpu.VMEM((1, _BCAP), jnp.int32),
            pltpu.VMEM((64, 64), jnp.float32),
            pltpu.SemaphoreType.DMA,
            pltpu.SemaphoreType.DMA,
            pltpu.SemaphoreType.DMA,
        ],
        compiler_params=pltpu.CompilerParams(needs_layout_passes=False),
    )
    def scan(idx_hbm, t3_hbm, tail_hbm, out_hbm, wpk, idxbuf, slab_a,
             slab_b, emb_v, bktv, bktp, bktp2, tail_v, sem_a, sem_b, sem_s):
        wid = lax.axis_index("s") * nc + lax.axis_index("c")
        lo = wid * _UPW
        hi = jnp.where(wid == _NW - 1, NUM_USERS, lo + _UPW)
        lanes = _I16()

        def _piece(p, cnt):
            pltpu.sync_copy(idx_hbm.at[pl.ds(p * 2048, 2048)], idxbuf)

            def _vstep(g, cnt):
                off = pl.multiple_of(g * 16, 16)
                v = idxbuf[pl.ds(off, 16)]
                m = (v >= lo) & (v < hi)
                cs = plsc.cumsum(m.astype(jnp.int32))
                tgt = cnt + cs - 1
                packed = lax.shift_left(v - lo, _POSB) | (
                    p * 2048 + off + lanes
                )
                plsc.store_scatter(wpk, [tgt], packed, mask=m)
                return cnt + jnp.sum(m.astype(jnp.int32))

            return lax.fori_loop(0, 128, _vstep, cnt)

        n_w = jnp.int32(0)
        for p in range(8):
            n_w = _piece(p, n_w)
        nv = (n_w + 15) // 16

        def _count_range(rlo, rhi):
            def _vstep(g, r):
                off = pl.multiple_of(g * 16, 16)
                rv = lax.shift_right_logical(wpk[pl.ds(off, 16)], _POSB)
                m = (rv >= rlo) & (rv < rhi) & (off + lanes < n_w)
                return r + jnp.sum(m.astype(jnp.int32))

            return lax.fori_loop(0, nv, _vstep, jnp.int32(0))

        def _fill_window(rlo, rhi, blk):
            for j in range(_BCAP // 16):
                bktv[pl.ds(j * 16, 16)] = jnp.full((16,), 0, jnp.int32) + rlo
                bktp[pl.ds(j * 16, 16)] = jnp.full((16,), -1, jnp.int32)
            wstart = blk * _BCAP

            def _vstep(g, carry):
                r, filled = carry
                off = pl.multiple_of(g * 16, 16)
                w = wpk[pl.ds(off, 16)]
                rv = lax.shift_right_logical(w, _POSB)
                pp = w & ((1 << _POSB) - 1)
                m = (rv >= rlo) & (rv < rhi) & (off + lanes < n_w)
                cs = plsc.cumsum(m.astype(jnp.int32))
                order = r + cs - 1
                keep = m & (order >= wstart) & (order < wstart + _BCAP)
                tgt = order - wstart
                plsc.store_scatter(bktv, [tgt], rv, mask=keep)
                plsc.store_scatter(bktp, [tgt], pp, mask=keep)
                return (
                    r + jnp.sum(m.astype(jnp.int32)),
                    filled + jnp.sum(keep.astype(jnp.int32)),
                )

            _, filled = lax.fori_loop(0, nv, _vstep, (jnp.int32(0), jnp.int32(0)))
            for j in range(_BCAP // 16):
                bktp2[0, pl.ds(j * 16, 16)] = bktp[pl.ds(j * 16, 16)]
            return filled

        def _scatter_out():
            pltpu.async_copy(
                emb_v,
                out_hbm.at[plsc.Indices(bktp2.at[0], ignored_value=-1)],
                sem_s,
            ).wait()

        def _process_slabs(u0, r0, filled):
            ng = (filled + 15) // 16
            cp = pltpu.async_copy(
                t3_hbm.at[0, :, pl.ds(u0, _CHW)], slab_a, sem_a
            )
            for a in range(8):
                buf = slab_a if a % 2 == 0 else slab_b
                if a < 7:
                    nxt_buf = slab_b if a % 2 == 0 else slab_a
                    nxt = pltpu.async_copy(
                        t3_hbm.at[a + 1, :, pl.ds(u0, _CHW)],
                        nxt_buf,
                        sem_b if a % 2 == 0 else sem_a,
                    )
                cp.wait()

                def _estep(g, _, buf=buf, a=a):
                    off = pl.multiple_of(g * 16, 16)
                    uv = bktv[pl.ds(off, 16)] - r0
                    sl = off + lanes
                    for b in range(8):
                        vals = plsc.load_gather(
                            buf, [jnp.full((16,), b, jnp.int32), uv]
                        )
                        plsc.store_scatter(
                            emb_v,
                            [sl, jnp.full((16,), a * 8 + b, jnp.int32)],
                            vals,
                        )
                    return 0

                lax.fori_loop(0, ng, _estep, 0)
                if a < 7:
                    cp = nxt

        def _chunk(c, _):
            u0 = jnp.minimum(lo + c * _CHW, _LAST_U0)
            r0 = u0 - lo
            m_c = _count_range(r0, r0 + _CHW)
            nblk = (m_c + _BCAP - 1) // _BCAP

            def _blk(blk, _):
                filled = _fill_window(r0, r0 + _CHW, blk)
                _process_slabs(u0, r0, filled)
                _scatter_out()
                return 0

            return lax.fori_loop(0, jnp.maximum(nblk, 1), _blk, 0)

        lax.fori_loop(0, _NCH, _chunk, jnp.int32(0))

        @pl.when(wid == _NW - 1)
        def _tail():
            pltpu.sync_copy(tail_hbm, tail_v)
            rt = _FULL_U - lo
            m_t = _count_range(rt, NUM_USERS - lo)
            nblk = (m_t + _BCAP - 1) // _BCAP

            def _blk(blk, _):
                filled = _fill_window(rt, NUM_USERS - lo, blk)
                ng = (filled + 15) // 16

                def _estep(g, _):
                    off = pl.multiple_of(g * 16, 16)
                    uv = bktv[pl.ds(off, 16)] - rt
                    sl = off + lanes
                    for f in range(FEAT_DIM):
                        vals = plsc.load_gather(
                            tail_v, [uv, jnp.full((16,), f, jnp.int32)]
                        )
                        plsc.store_scatter(
                            emb_v, [sl, jnp.full((16,), f, jnp.int32)], vals
                        )
                    return 0

                lax.fori_loop(0, ng, _estep, 0)
                _scatter_out()
                return 0

            lax.fori_loop(0, nblk, _blk, 0)

    return scan


def _layernorm(x, eps=1e-5):
    mu = jnp.mean(x, axis=-1, keepdims=True)
    var = jnp.mean((x - mu) ** 2, axis=-1, keepdims=True)
    return (x - mu) / jnp.sqrt(var + eps)


def _gelu_exact(x):
    return 0.5 * x * (1.0 + lax.erf(x * 0.7071067811865476))


def _mlp_body(emb_ref, w1_ref, b1_ref, w2_ref, b2_ref, w3_ref, b3_ref, out_ref):
    x = emb_ref[...][:, :FEAT_DIM]
    h = jnp.dot(x, w1_ref[...], preferred_element_type=jnp.float32) + b1_ref[...]
    h = _gelu_exact(_layernorm(h))
    h = jnp.dot(h, w2_ref[...], preferred_element_type=jnp.float32) + b2_ref[...]
    h = _gelu_exact(_layernorm(h))
    h = jnp.dot(h, w3_ref[...], preferred_element_type=jnp.float32) + b3_ref[...]
    out_ref[...] = _gelu_exact(h)


def _tc_mlp(emb, w1, b1, w2, b2, w3, b3, blk=2048, interpret=False):
    batch = emb.shape[0]
    grid = (batch // blk,)
    rep2 = lambda i: (0, 0)
    return pl.pallas_call(
        _mlp_body,
        grid=grid,
        in_specs=[
            pl.BlockSpec((blk, emb.shape[1]), lambda i: (i, 0)),
            pl.BlockSpec(w1.shape, rep2),
            pl.BlockSpec(b1.shape, rep2),
            pl.BlockSpec(w2.shape, rep2),
            pl.BlockSpec(b2.shape, rep2),
            pl.BlockSpec(w3.shape, rep2),
            pl.BlockSpec(b3.shape, rep2),
        ],
        out_specs=pl.BlockSpec((blk, w3.shape[1]), lambda i: (i, 0)),
        out_shape=jax.ShapeDtypeStruct((batch, w3.shape[1]), jnp.float32),
        interpret=interpret,
    )(emb, w1, b1, w2, b2, w3, b3)


def kernel(user_ids, table, W1, b1, W2, b2, W3, b3):
    idx = user_ids.astype(jnp.int32)
    t3 = table.T.reshape(8, 8, NUM_USERS)
    tail = lax.slice(table, (_FULL_U, 0), (NUM_USERS, FEAT_DIM))
    emb = _make_sc_scan()(idx, t3, tail)
    return _tc_mlp(
        emb,
        W1,
        b1.reshape(1, -1),
        W2,
        b2.reshape(1, -1),
        W3,
        b3.reshape(1, -1),
    )


# speedup vs baseline: 3.5128x; 1.0325x over previous
import functools

import jax
import jax.numpy as jnp
from jax import lax
from jax.experimental import pallas as pl
from jax.experimental.pallas import tpu as pltpu
from jax.experimental.pallas import tpu_sc as plsc

BATCH = 16384
FEAT_DIM = 64
NUM_USERS = 1000000

_NW = 32
_K = 32
_CHW = _K * 128
_TPW = 245
_UPW = _TPW * 128
_NCH = 8
_FULL_U = 999936
_LAST_U0 = _FULL_U - _CHW
_BCAP = 128
_POSB = 14
_I16 = lambda: lax.iota(jnp.int32, 16)


@functools.cache
def _make_sc_scan():
    mesh = plsc.VectorSubcoreMesh(core_axis_name="c", subcore_axis_name="s")
    nc = 2

    @functools.partial(
        pl.kernel,
        mesh=mesh,
        out_type=jax.ShapeDtypeStruct((BATCH, 128), jnp.float32),
        scratch_types=[
            pltpu.VMEM((BATCH,), jnp.int32),
            pltpu.VMEM((BATCH,), jnp.int32),
            pltpu.VMEM((8, _CHW), jnp.float32),
            pltpu.VMEM((8, _CHW), jnp.float32),
            pltpu.VMEM((_BCAP, 128), jnp.float32),
            pltpu.VMEM((_BCAP,), jnp.int32),
            pltpu.VMEM((_BCAP,), jnp.int32),
            pltpu.VMEM((1, _BCAP), jnp.int32),
            pltpu.VMEM((64, 64), jnp.float32),
            pltpu.SemaphoreType.DMA,
            pltpu.SemaphoreType.DMA,
            pltpu.SemaphoreType.DMA,
        ],
        compiler_params=pltpu.CompilerParams(needs_layout_passes=False),
    )
    def scan(idx_hbm, t3_hbm, tail_hbm, out_hbm, wpk, idxbuf, slab_a,
             slab_b, emb_v, bktv, bktp, bktp2, tail_v, sem_a, sem_b, sem_s):
        wid = lax.axis_index("s") * nc + lax.axis_index("c")
        lo = wid * _UPW
        hi = jnp.where(wid == _NW - 1, NUM_USERS, lo + _UPW)
        lanes = _I16()

        pltpu.async_copy(t3_hbm.at[0, :, pl.ds(lo, _CHW)], slab_a, sem_a)
        pltpu.sync_copy(idx_hbm, idxbuf)

        def _vstep1(g, cnt):
            off = pl.multiple_of(g * 16, 16)
            v = idxbuf[pl.ds(off, 16)]
            m = (v >= lo) & (v < hi)
            cs = plsc.cumsum(m.astype(jnp.int32))
            tgt = cnt + cs - 1
            packed = lax.shift_left(v - lo, _POSB) | (off + lanes)
            plsc.store_scatter(wpk, [tgt], packed, mask=m)
            return cnt + jnp.sum(m.astype(jnp.int32))

        n_w = lax.fori_loop(0, BATCH // 16, _vstep1, jnp.int32(0))
        nv = (n_w + 15) // 16

        def _count_range(rlo, rhi):
            def _vstep(g, r):
                off = pl.multiple_of(g * 16, 16)
                rv = lax.shift_right_logical(wpk[pl.ds(off, 16)], _POSB)
                m = (rv >= rlo) & (rv < rhi) & (off + lanes < n_w)
                return r + jnp.sum(m.astype(jnp.int32))

            return lax.fori_loop(0, nv, _vstep, jnp.int32(0))

        def _fill_window(rlo, rhi, blk):
            for j in range(_BCAP // 16):
                bktv[pl.ds(j * 16, 16)] = jnp.full((16,), 0, jnp.int32) + rlo
                bktp[pl.ds(j * 16, 16)] = jnp.full((16,), -1, jnp.int32)
            wstart = blk * _BCAP

            def _vstep(g, carry):
                r, filled = carry
                off = pl.multiple_of(g * 16, 16)
                w = wpk[pl.ds(off, 16)]
                rv = lax.shift_right_logical(w, _POSB)
                pp = w & ((1 << _POSB) - 1)
                m = (rv >= rlo) & (rv < rhi) & (off + lanes < n_w)
                cs = plsc.cumsum(m.astype(jnp.int32))
                order = r + cs - 1
                keep = m & (order >= wstart) & (order < wstart + _BCAP)
                tgt = order - wstart
                plsc.store_scatter(bktv, [tgt], rv, mask=keep)
                plsc.store_scatter(bktp, [tgt], pp, mask=keep)
                return (
                    r + jnp.sum(m.astype(jnp.int32)),
                    filled + jnp.sum(keep.astype(jnp.int32)),
                )

            _, filled = lax.fori_loop(0, nv, _vstep, (jnp.int32(0), jnp.int32(0)))
            for j in range(_BCAP // 16):
                bktp2[0, pl.ds(j * 16, 16)] = bktp[pl.ds(j * 16, 16)]
            return filled

        def _scatter_out():
            pltpu.async_copy(
                emb_v,
                out_hbm.at[plsc.Indices(bktp2.at[0], ignored_value=-1)],
                sem_s,
            ).wait()

        def _extract(buf, a, r0, ng):
            def _estep(g, _, buf=buf, a=a):
                off = pl.multiple_of(g * 16, 16)
                uv = bktv[pl.ds(off, 16)] - r0
                sl = off + lanes
                for b in range(8):
                    vals = plsc.load_gather(
                        buf, [jnp.full((16,), b, jnp.int32), uv]
                    )
                    plsc.store_scatter(
                        emb_v,
                        [sl, jnp.full((16,), a * 8 + b, jnp.int32)],
                        vals,
                    )
                return 0

            lax.fori_loop(0, ng, _estep, 0)

        def _slab_wait(buf, sem):
            pltpu.make_async_copy(
                t3_hbm.at[0, :, pl.ds(lo, _CHW)], buf, sem
            ).wait()

        def _process_slabs_chained(u0, u_next, r0, filled):
            ng = (filled + 15) // 16
            for a in range(8):
                buf = slab_a if a % 2 == 0 else slab_b
                cur_sem = sem_a if a % 2 == 0 else sem_b
                nxt_buf = slab_b if a % 2 == 0 else slab_a
                nxt_sem = sem_b if a % 2 == 0 else sem_a
                if a < 7:
                    pltpu.async_copy(
                        t3_hbm.at[a + 1, :, pl.ds(u0, _CHW)], nxt_buf, nxt_sem
                    )
                else:
                    pltpu.async_copy(
                        t3_hbm.at[0, :, pl.ds(u_next, _CHW)], nxt_buf, nxt_sem
                    )
                _slab_wait(buf, cur_sem)
                _extract(buf, a, r0, ng)

        def _process_slabs_single(u0, r0, filled):
            ng = (filled + 15) // 16
            for a in range(8):
                cp = pltpu.async_copy(
                    t3_hbm.at[a, :, pl.ds(u0, _CHW)], slab_b, sem_b
                )
                cp.wait()
                _extract(slab_b, a, r0, ng)

        def _chunk(c, _):
            u0 = jnp.minimum(lo + c * _CHW, _LAST_U0)
            u_next = jnp.minimum(lo + (c + 1) * _CHW, _LAST_U0)
            r0 = u0 - lo
            m_c = _count_range(r0, r0 + _CHW)
            filled = _fill_window(r0, r0 + _CHW, 0)
            _process_slabs_chained(u0, u_next, r0, filled)
            _scatter_out()
            nblk = (m_c + _BCAP - 1) // _BCAP

            def _rare(blk, _):
                filled = _fill_window(r0, r0 + _CHW, blk)
                _process_slabs_single(u0, r0, filled)
                _scatter_out()
                return 0

            return lax.fori_loop(1, nblk, _rare, jnp.int32(0))

        lax.fori_loop(0, _NCH, _chunk, jnp.int32(0))
        _slab_wait(slab_a, sem_a)

        @pl.when(wid == _NW - 1)
        def _tail():
            pltpu.sync_copy(tail_hbm, tail_v)
            rt = _FULL_U - lo
            m_t = _count_range(rt, NUM_USERS - lo)
            nblk = (m_t + _BCAP - 1) // _BCAP

            def _blk(blk, _):
                filled = _fill_window(rt, NUM_USERS - lo, blk)
                ng = (filled + 15) // 16

                def _estep(g, _):
                    off = pl.multiple_of(g * 16, 16)
                    uv = bktv[pl.ds(off, 16)] - rt
                    sl = off + lanes
                    for f in range(FEAT_DIM):
                        vals = plsc.load_gather(
                            tail_v, [uv, jnp.full((16,), f, jnp.int32)]
                        )
                        plsc.store_scatter(
                            emb_v, [sl, jnp.full((16,), f, jnp.int32)], vals
                        )
                    return 0

                lax.fori_loop(0, ng, _estep, 0)
                _scatter_out()
                return 0

            lax.fori_loop(0, nblk, _blk, 0)

    return scan


def _layernorm(x, eps=1e-5):
    mu = jnp.mean(x, axis=-1, keepdims=True)
    var = jnp.mean((x - mu) ** 2, axis=-1, keepdims=True)
    return (x - mu) / jnp.sqrt(var + eps)


def _gelu_exact(x):
    return 0.5 * x * (1.0 + lax.erf(x * 0.7071067811865476))


def _mlp_body(emb_ref, w1_ref, b1_ref, w2_ref, b2_ref, w3_ref, b3_ref, out_ref):
    x = emb_ref[...][:, :FEAT_DIM]
    h = jnp.dot(x, w1_ref[...], preferred_element_type=jnp.float32) + b1_ref[...]
    h = _gelu_exact(_layernorm(h))
    h = jnp.dot(h, w2_ref[...], preferred_element_type=jnp.float32) + b2_ref[...]
    h = _gelu_exact(_layernorm(h))
    h = jnp.dot(h, w3_ref[...], preferred_element_type=jnp.float32) + b3_ref[...]
    out_ref[...] = _gelu_exact(h)


def _tc_mlp(emb, w1, b1, w2, b2, w3, b3, blk=4096, interpret=False):
    batch = emb.shape[0]
    grid = (batch // blk,)
    rep2 = lambda i: (0, 0)
    return pl.pallas_call(
        _mlp_body,
        grid=grid,
        in_specs=[
            pl.BlockSpec((blk, emb.shape[1]), lambda i: (i, 0)),
            pl.BlockSpec(w1.shape, rep2),
            pl.BlockSpec(b1.shape, rep2),
            pl.BlockSpec(w2.shape, rep2),
            pl.BlockSpec(b2.shape, rep2),
            pl.BlockSpec(w3.shape, rep2),
            pl.BlockSpec(b3.shape, rep2),
        ],
        out_specs=pl.BlockSpec((blk, w3.shape[1]), lambda i: (i, 0)),
        out_shape=jax.ShapeDtypeStruct((batch, w3.shape[1]), jnp.float32),
        interpret=interpret,
    )(emb, w1, b1, w2, b2, w3, b3)


def kernel(user_ids, table, W1, b1, W2, b2, W3, b3):
    idx = user_ids.astype(jnp.int32)
    t3 = table.T.reshape(8, 8, NUM_USERS)
    tail = lax.slice(table, (_FULL_U, 0), (NUM_USERS, FEAT_DIM))
    emb = _make_sc_scan()(idx, t3, tail)
    return _tc_mlp(
        emb,
        W1,
        b1.reshape(1, -1),
        W2,
        b2.reshape(1, -1),
        W3,
        b3.reshape(1, -1),
    )
